# baseline (speedup 1.0000x reference)
import functools

import jax
import jax.numpy as jnp
from jax import lax
from jax.experimental import pallas as pl
from jax.experimental.pallas import tpu as pltpu

_N, _T, _F = 1024, 512, 256
_B = 8
_NB = 8


def _body(ptr_ref, len_ref, x_ref, q_ref, o_ref):
    i = pl.program_id(0)
    q = q_ref[0]
    outs = []
    for k in range(_NB):
        n = i * _NB + k
        g = jnp.int32(0)
        for j in range(1, _B):
            g += jnp.where(ptr_ref[j] <= n, 1, 0).astype(jnp.int32)
        node_len = len_ref[g]
        x = x_ref[k]
        s = jax.lax.dot_general(
            x, q.reshape(_F, 1),
            dimension_numbers=(((1,), (0,)), ((), ())),
            preferred_element_type=jnp.float32,
        )
        t_idx = lax.broadcasted_iota(jnp.int32, (_T, 1), 0)
        w = jnp.where(t_idx < node_len, s, 0.0)
        out = jax.lax.dot_general(
            w, x,
            dimension_numbers=(((0,), (0,)), ((), ())),
            preferred_element_type=jnp.float32,
        )
        outs.append(out)
    o_ref[...] = jnp.concatenate(outs, axis=0)


def kernel(nodes_output, ptr, lengths, Wq_w):
    ptr_i = ptr.astype(jnp.int32)
    len_i = lengths.astype(jnp.int32)
    q2 = Wq_w.reshape(1, _F)
    grid_spec = pltpu.PrefetchScalarGridSpec(
        num_scalar_prefetch=2,
        grid=(_N // _NB,),
        in_specs=[
            pl.BlockSpec((_NB, _T, _F), lambda i, p, l: (i, 0, 0)),
            pl.BlockSpec((1, _F), lambda i, p, l: (0, 0)),
        ],
        out_specs=pl.BlockSpec((_NB, _F), lambda i, p, l: (i, 0)),
    )
    return pl.pallas_call(
        _body,
        grid_spec=grid_spec,
        out_shape=jax.ShapeDtypeStruct((_N, _F), jnp.float32),
    )(ptr_i, len_i, nodes_output, q2)
